# baseline (speedup 1.0000x reference)
import jax
import jax.numpy as jnp
from jax import lax
from jax.experimental import pallas as pl
from jax.experimental.pallas import tpu as pltpu
from jax.experimental.pallas import tpu_sc as plsc

NC, NS, L = 2, 16, 16
NW = NC * NS
H = W = 2048
N = H * W
R_TC = 1920
RB = 64
SC_PX0 = R_TC * W
SC_PX = N - SC_PX0
PER_W = SC_PX // NW
B = 1024
ITERS = PER_W // B
NBUF = 4


def _tc_body(xref, oref):
    m = xref[0]
    idx = jnp.zeros((RB, W), jnp.int32)
    for c in range(1, 8):
        vc = xref[c]
        gt = vc > m
        m = jnp.where(gt, vc, m)
        idx = jnp.where(gt, jnp.full((RB, W), c, jnp.int32), idx)
    dx = xref[8]
    dy = xref[9]
    mag = jnp.sqrt(dx * dx + dy * dy)
    zero = jnp.zeros((RB, W), jnp.float32)
    for c in range(8):
        oref[c] = jnp.where(idx == c, mag, zero)


def _sc_compute_block(xb, ob):
    @plsc.parallel_loop(0, B // L, step=1, unroll=4)
    def grp(g):
        sl = pl.ds(g * L, L)
        m = xb[0, sl]
        idx = jnp.zeros((L,), jnp.int32)
        for c in range(1, 8):
            vc = xb[c, sl]
            gt = vc > m
            m = jnp.where(gt, vc, m)
            idx = jnp.where(gt, jnp.full((L,), c, jnp.int32), idx)
        dx = xb[8, sl]
        dy = xb[9, sl]
        s2 = dx * dx + dy * dy
        s2s = jnp.maximum(s2, jnp.full((L,), 1e-30, jnp.float32))
        ii = lax.bitcast_convert_type(s2s, jnp.int32)
        seed = jnp.full((L,), 0x5F3759DF, jnp.int32) - (ii >> 1)
        y = lax.bitcast_convert_type(seed, jnp.float32)
        half_s = s2s * 0.5
        for _ in range(3):
            y = y * (1.5 - half_s * y * y)
        mag = s2 * y
        zero = jnp.zeros((L,), jnp.float32)
        for c in range(8):
            ob[c, sl] = jnp.where(idx == c, mag, zero)


def _sc_body(x_hbm, out_hbm, xbuf, obuf, *sems):
    isems = sems[:NBUF]
    osems = sems[NBUF:]
    cid = lax.axis_index("c")
    sid = lax.axis_index("s")
    wid = sid * NC + cid
    base = SC_PX0 + wid * PER_W
    obase = wid * PER_W

    for k in range(NBUF):
        pltpu.async_copy(
            x_hbm.at[:, pl.ds(base + k * B, B)], xbuf.at[k], isems[k])

    def outer(jj, carry):
        for k in range(NBUF):
            i = jj * NBUF + k
            pltpu.make_async_copy(
                x_hbm.at[:, pl.ds(base + i * B, B)], xbuf.at[k],
                isems[k]).wait()

            @pl.when(jj > 0)
            def _():
                pltpu.make_async_copy(
                    obuf.at[k], out_hbm.at[:, pl.ds(obase, B)],
                    osems[k]).wait()

            _sc_compute_block(xbuf.at[k], obuf.at[k])

            @pl.when(i + NBUF < ITERS)
            def _():
                pltpu.async_copy(
                    x_hbm.at[:, pl.ds(base + (i + NBUF) * B, B)], xbuf.at[k],
                    isems[k])

            pltpu.async_copy(
                obuf.at[k], out_hbm.at[:, pl.ds(obase + i * B, B)], osems[k])
        return carry

    lax.fori_loop(0, ITERS // NBUF, outer, 0, unroll=False)

    for k in range(NBUF):
        pltpu.make_async_copy(
            obuf.at[k], out_hbm.at[:, pl.ds(obase, B)], osems[k]).wait()


@jax.jit
def _run(x3):
    xf = x3.reshape(10, N)
    sc_f = pl.kernel(
        _sc_body,
        out_type=jax.ShapeDtypeStruct((8, SC_PX), jnp.float32),
        mesh=plsc.VectorSubcoreMesh(
            core_axis_name="c", subcore_axis_name="s",
            num_cores=NC, num_subcores=NS,
        ),
        scratch_types=[
            pltpu.VMEM((NBUF, 10, B), jnp.float32),
            pltpu.VMEM((NBUF, 8, B), jnp.float32),
        ] + [pltpu.SemaphoreType.DMA] * (2 * NBUF),
    )
    sc_out = sc_f(xf)
    tc_out = pl.pallas_call(
        _tc_body,
        grid=(R_TC // RB,),
        in_specs=[pl.BlockSpec((10, RB, W), lambda i: (0, i, 0))],
        out_specs=pl.BlockSpec((8, RB, W), lambda i: (0, i, 0)),
        out_shape=jax.ShapeDtypeStruct((8, H, W), jnp.float32),
        compiler_params=pltpu.CompilerParams(
            dimension_semantics=("arbitrary",)),
    )(x3)
    out = lax.dynamic_update_slice(
        tc_out.reshape(8, N), sc_out, (0, SC_PX0))
    return out


def kernel(x):
    out = _run(x.reshape(10, H, W))
    return out.reshape(1, 8, H, W)


# speedup vs baseline: 5.3276x; 5.3276x over previous
import jax
import jax.numpy as jnp
from jax import lax
from jax.experimental import pallas as pl
from jax.experimental.pallas import tpu as pltpu
from jax.experimental.pallas import tpu_sc as plsc

NC, NS, L = 2, 16, 16
NW = NC * NS
H = W = 2048
R_TC = 1920
RB = 64
SC_ROWS = H - R_TC
ROWS_PER_W = SC_ROWS // NW
NBUF = 2


def _tc_body(xref, oref):
    m = xref[0]
    idx = jnp.zeros((RB, W), jnp.int32)
    for c in range(1, 8):
        vc = xref[c]
        gt = vc > m
        m = jnp.where(gt, vc, m)
        idx = jnp.where(gt, jnp.full((RB, W), c, jnp.int32), idx)
    dx = xref[8]
    dy = xref[9]
    mag = jnp.sqrt(dx * dx + dy * dy)
    zero = jnp.zeros((RB, W), jnp.float32)
    for c in range(8):
        oref[c] = jnp.where(idx == c, mag, zero)


def _sc_compute_row(xb, ob):
    @plsc.parallel_loop(0, W // L, step=1, unroll=4)
    def grp(g):
        sl = pl.ds(g * L, L)
        m = xb[0, 0, sl]
        idx = jnp.zeros((L,), jnp.int32)
        for c in range(1, 8):
            vc = xb[c, 0, sl]
            gt = vc > m
            m = jnp.where(gt, vc, m)
            idx = jnp.where(gt, jnp.full((L,), c, jnp.int32), idx)
        dx = xb[8, 0, sl]
        dy = xb[9, 0, sl]
        s2 = dx * dx + dy * dy
        s2s = jnp.maximum(s2, jnp.full((L,), 1e-30, jnp.float32))
        ii = lax.bitcast_convert_type(s2s, jnp.int32)
        seed = jnp.full((L,), 0x5F3759DF, jnp.int32) - (ii >> 1)
        y = lax.bitcast_convert_type(seed, jnp.float32)
        half_s = s2s * 0.5
        for _ in range(3):
            y = y * (1.5 - half_s * y * y)
        mag = s2 * y
        zero = jnp.zeros((L,), jnp.float32)
        for c in range(8):
            ob[c, 0, sl] = jnp.where(idx == c, mag, zero)


def _sc_body(x_hbm, out_hbm, xbuf, obuf, *sems):
    isems = sems[:NBUF]
    osems = sems[NBUF:]
    cid = lax.axis_index("c")
    sid = lax.axis_index("s")
    wid = sid * NC + cid
    row0 = R_TC + wid * ROWS_PER_W
    orow0 = wid * ROWS_PER_W

    for k in range(NBUF):
        pltpu.async_copy(
            x_hbm.at[:, pl.ds(row0 + k, 1), :], xbuf.at[k], isems[k])

    def outer(jj, carry):
        for k in range(NBUF):
            i = jj * NBUF + k
            pltpu.make_async_copy(
                x_hbm.at[:, pl.ds(row0 + i, 1), :], xbuf.at[k],
                isems[k]).wait()

            @pl.when(jj > 0)
            def _():
                pltpu.make_async_copy(
                    obuf.at[k], out_hbm.at[:, pl.ds(orow0, 1), :],
                    osems[k]).wait()

            _sc_compute_row(xbuf.at[k], obuf.at[k])

            @pl.when(i + NBUF < ROWS_PER_W)
            def _():
                pltpu.async_copy(
                    x_hbm.at[:, pl.ds(row0 + i + NBUF, 1), :], xbuf.at[k],
                    isems[k])

            pltpu.async_copy(
                obuf.at[k], out_hbm.at[:, pl.ds(orow0 + i, 1), :], osems[k])
        return carry

    lax.fori_loop(0, ROWS_PER_W // NBUF, outer, 0, unroll=False)

    for k in range(NBUF):
        pltpu.make_async_copy(
            obuf.at[k], out_hbm.at[:, pl.ds(orow0, 1), :], osems[k]).wait()


@jax.jit
def _run(x3):
    sc_f = pl.kernel(
        _sc_body,
        out_type=jax.ShapeDtypeStruct((8, SC_ROWS, W), jnp.float32),
        mesh=plsc.VectorSubcoreMesh(
            core_axis_name="c", subcore_axis_name="s",
            num_cores=NC, num_subcores=NS,
        ),
        scratch_types=[
            pltpu.VMEM((NBUF, 10, 1, W), jnp.float32),
            pltpu.VMEM((NBUF, 8, 1, W), jnp.float32),
        ] + [pltpu.SemaphoreType.DMA] * (2 * NBUF),
    )
    sc_out = sc_f(x3)
    tc_out = pl.pallas_call(
        _tc_body,
        grid=(R_TC // RB,),
        in_specs=[pl.BlockSpec((10, RB, W), lambda i: (0, i, 0))],
        out_specs=pl.BlockSpec((8, RB, W), lambda i: (0, i, 0)),
        out_shape=jax.ShapeDtypeStruct((8, H, W), jnp.float32),
        compiler_params=pltpu.CompilerParams(
            dimension_semantics=("arbitrary",)),
    )(x3)
    return lax.dynamic_update_slice(tc_out, sc_out, (0, R_TC, 0))


def kernel(x):
    out = _run(x.reshape(10, H, W))
    return out.reshape(1, 8, H, W)
